# baseline (speedup 1.0000x reference)
import functools

import jax
import jax.numpy as jnp
from jax import lax
from jax.experimental import pallas as pl
from jax.experimental.pallas import tpu as pltpu
from jax.experimental.pallas import tpu_sc as plsc

D = 32
LANES = 16
NW = 32


def _scale_table(table_t):
    v = table_t.shape[1]
    block = 12800

    def sk(x_ref, o_ref):
        x = x_ref[...]
        ssq = jnp.sum(x * x, axis=0, keepdims=True)
        o_ref[...] = x * lax.rsqrt(jnp.maximum(ssq, 1e-24))

    return pl.pallas_call(
        sk,
        grid=(pl.cdiv(v, block),),
        in_specs=[pl.BlockSpec((D, block), lambda i: (0, i))],
        out_specs=pl.BlockSpec((D, block), lambda i: (0, i)),
        out_shape=jax.ShapeDtypeStruct((D, v), jnp.float32),
    )(table_t)


NT = 50
CHUNK = NT * 128
TUNROLL = 5


@functools.cache
def _gather_soa(e, v):
    etiles = e // 128
    per_half = etiles // NT
    n_chunks = 2 * per_half
    assert e % 128 == 0 and etiles % NT == 0
    mesh = plsc.VectorSubcoreMesh(core_axis_name="c", subcore_axis_name="s")

    @functools.partial(
        pl.kernel,
        out_type=jax.ShapeDtypeStruct((2, D // 8, etiles, 8, 128), jnp.float32),
        mesh=mesh,
        scratch_types=[
            pltpu.VMEM((v,), jnp.float32),
            pltpu.VMEM((CHUNK,), jnp.int32),
            pltpu.VMEM((CHUNK,), jnp.int32),
            pltpu.VMEM((NT, 128), jnp.float32),
            pltpu.VMEM((NT, 128), jnp.float32),
            pltpu.VMEM_SHARED((2, CHUNK), jnp.int32),
            pltpu.SemaphoreType.DMA,
            pltpu.SemaphoreType.DMA,
            pltpu.SemaphoreType.DMA,
            pltpu.SemaphoreType.DMA,
            pltpu.SemaphoreType.DMA,
            pltpu.SemaphoreType.DMA,
        ],
        compiler_params=pltpu.CompilerParams(
            use_tc_tiling_on_sc=False, needs_layout_passes=False
        ),
    )
    def k(idx_hbm, st_hbm, out_hbm, trow, i0, i1, o0, o1, sp_idx,
          si0, si1, sw0, sw1, sp0, sp1):
        idx_v = (i0, i1)
        out_v = (o0, o1)
        sem_i = (si0, si1)
        sem_w = (sw0, sw1)
        sem_sp = (sp0, sp1)
        sid = lax.axis_index("s")
        d = lax.axis_index("c") * (NW // 2) + sid
        dhi = d // 8
        dlo = d % 8

        def sp_copy(i, b):
            return pltpu.make_async_copy(
                idx_hbm.at[pl.ds(i * CHUNK, CHUNK)], sp_idx.at[b], sem_sp[b])

        def idx_copy(b):
            return pltpu.make_async_copy(
                sp_idx.at[b], idx_v[b], sem_i[b])

        def wb_copy(i, b):
            s = i // per_half
            t0 = (i % per_half) * NT
            return pltpu.make_async_copy(
                out_v[b],
                out_hbm.at[s, dhi, pl.ds(t0, NT), dlo, :],
                sem_w[b])

        def compute(b):
            @plsc.parallel_loop(0, NT, unroll=TUNROLL)
            def _(t):
                for u in range(128 // LANES):
                    j = u * LANES
                    idx16 = idx_v[b][pl.ds(t * 128 + j, LANES)]
                    out_v[b][t, pl.ds(j, LANES)] = plsc.load_gather(
                        trow, [idx16])

        pltpu.sync_copy(st_hbm.at[d], trow)

        @pl.when(sid == 0)
        def _():
            sp_copy(0, 0).start()
            sp_copy(1, 1).start()

        def superstep(ss, carry):
            for b in (0, 1):
                i = 2 * ss + b

                @pl.when(sid == 0)
                def _():
                    sp_copy(i, b).wait()

                plsc.subcore_barrier()
                idx_copy(b).start()
                idx_copy(b).wait()
                plsc.subcore_barrier()

                @pl.when(jnp.logical_and(sid == 0, i + 2 < n_chunks))
                def _():
                    sp_copy(i + 2, b).start()

                @pl.when(i >= 2)
                def _():
                    wb_copy(i - 2, b).wait()

                compute(b)
                wb_copy(i, b).start()

            return carry

        lax.fori_loop(0, n_chunks // 2, superstep, 0)
        wb_copy(n_chunks - 2, 0).wait()
        wb_copy(n_chunks - 1, 1).wait()

    return k


def _rel_proj_t(attr_t, r_weight):
    e = attr_t.shape[1]
    block = 12800

    def mm(w_ref, x_ref, o_ref):
        o_ref[...] = lax.dot_general(
            w_ref[...], x_ref[...],
            (((1,), (0,)), ((), ())),
            preferred_element_type=jnp.float32,
        )

    return pl.pallas_call(
        mm,
        grid=(e // block,),
        in_specs=[
            pl.BlockSpec((D, 16), lambda i: (0, 0)),
            pl.BlockSpec((16, block), lambda i: (0, i)),
        ],
        out_specs=pl.BlockSpec((D, block), lambda i: (0, i)),
        out_shape=jax.ShapeDtypeStruct((D, e), jnp.float32),
    )(r_weight, attr_t)


def kernel(edge_index, edge_attr, entity_table, r_weight):
    e = edge_index.shape[1]
    v = entity_table.shape[0]
    idx_flat = edge_index.reshape(2 * e)
    st = _scale_table(entity_table.T)
    out5 = _gather_soa(e, v)(idx_flat, st)
    edge_emb = jnp.transpose(out5, (0, 2, 4, 1, 3)).reshape(2, e, D)
    rel_emb = _rel_proj_t(edge_attr.T, r_weight).T
    return (edge_emb, rel_emb)


# speedup vs baseline: 1.0292x; 1.0292x over previous
import functools

import jax
import jax.numpy as jnp
from jax import lax
from jax.experimental import pallas as pl
from jax.experimental.pallas import tpu as pltpu
from jax.experimental.pallas import tpu_sc as plsc

D = 32
LANES = 16
NW = 32


def _scale_table(table_t):
    v = table_t.shape[1]
    block = 12800

    def sk(x_ref, o_ref):
        x = x_ref[...]
        ssq = jnp.sum(x * x, axis=0, keepdims=True)
        o_ref[...] = x * lax.rsqrt(jnp.maximum(ssq, 1e-24))

    return pl.pallas_call(
        sk,
        grid=(pl.cdiv(v, block),),
        in_specs=[pl.BlockSpec((D, block), lambda i: (0, i))],
        out_specs=pl.BlockSpec((D, block), lambda i: (0, i)),
        out_shape=jax.ShapeDtypeStruct((D, v), jnp.float32),
    )(table_t)


NT = 50
CHUNK = NT * 128
TUNROLL = 10


@functools.cache
def _gather_soa(e, v):
    etiles = e // 128
    per_half = etiles // NT
    n_chunks = 2 * per_half
    assert e % 128 == 0 and etiles % NT == 0
    mesh = plsc.VectorSubcoreMesh(core_axis_name="c", subcore_axis_name="s")

    @functools.partial(
        pl.kernel,
        out_type=jax.ShapeDtypeStruct((2, D // 8, etiles, 8, 128), jnp.float32),
        mesh=mesh,
        scratch_types=[
            pltpu.VMEM((v,), jnp.float32),
            pltpu.VMEM((CHUNK,), jnp.int32),
            pltpu.VMEM((CHUNK,), jnp.int32),
            pltpu.VMEM((NT, 128), jnp.float32),
            pltpu.VMEM((NT, 128), jnp.float32),
            pltpu.SemaphoreType.DMA,
            pltpu.SemaphoreType.DMA,
            pltpu.SemaphoreType.DMA,
            pltpu.SemaphoreType.DMA,
        ],
        compiler_params=pltpu.CompilerParams(
            use_tc_tiling_on_sc=False, needs_layout_passes=False
        ),
    )
    def k(idx_hbm, st_hbm, out_hbm, trow, i0, i1, o0, o1, si0, si1, sw0, sw1):
        idx_v = (i0, i1)
        out_v = (o0, o1)
        sem_i = (si0, si1)
        sem_w = (sw0, sw1)
        d = lax.axis_index("c") * (NW // 2) + lax.axis_index("s")
        dhi = d // 8
        dlo = d % 8

        def idx_copy(i, b):
            return pltpu.make_async_copy(
                idx_hbm.at[pl.ds(i * CHUNK, CHUNK)], idx_v[b], sem_i[b])

        def wb_copy(i, b):
            s = i // per_half
            t0 = (i % per_half) * NT
            return pltpu.make_async_copy(
                out_v[b],
                out_hbm.at[s, dhi, pl.ds(t0, NT), dlo, :],
                sem_w[b])

        def compute(b):
            @plsc.parallel_loop(0, NT, unroll=TUNROLL)
            def _(t):
                for u in range(128 // LANES):
                    j = u * LANES
                    idx16 = idx_v[b][pl.ds(t * 128 + j, LANES)]
                    out_v[b][t, pl.ds(j, LANES)] = plsc.load_gather(
                        trow, [idx16])

        pltpu.sync_copy(st_hbm.at[d], trow)
        idx_copy(0, 0).start()
        idx_copy(1, 1).start()

        def superstep(ss, carry):
            for b in (0, 1):
                i = 2 * ss + b
                idx_copy(i, b).wait()

                @pl.when(i >= 2)
                def _():
                    wb_copy(i - 2, b).wait()

                compute(b)
                wb_copy(i, b).start()

                @pl.when(i + 2 < n_chunks)
                def _():
                    idx_copy(i + 2, b).start()

            return carry

        lax.fori_loop(0, n_chunks // 2, superstep, 0)
        wb_copy(n_chunks - 2, 0).wait()
        wb_copy(n_chunks - 1, 1).wait()

    return k


def _rel_proj_t(attr_t, r_weight):
    e = attr_t.shape[1]
    block = 12800

    def mm(w_ref, x_ref, o_ref):
        o_ref[...] = lax.dot_general(
            w_ref[...], x_ref[...],
            (((1,), (0,)), ((), ())),
            preferred_element_type=jnp.float32,
        )

    return pl.pallas_call(
        mm,
        grid=(e // block,),
        in_specs=[
            pl.BlockSpec((D, 16), lambda i: (0, 0)),
            pl.BlockSpec((16, block), lambda i: (0, i)),
        ],
        out_specs=pl.BlockSpec((D, block), lambda i: (0, i)),
        out_shape=jax.ShapeDtypeStruct((D, e), jnp.float32),
    )(r_weight, attr_t)


def kernel(edge_index, edge_attr, entity_table, r_weight):
    e = edge_index.shape[1]
    v = entity_table.shape[0]
    idx_flat = edge_index.reshape(2 * e)
    st = _scale_table(entity_table.T)
    out5 = _gather_soa(e, v)(idx_flat, st)
    edge_emb = jnp.transpose(out5, (0, 2, 4, 1, 3)).reshape(2, e, D)
    rel_emb = _rel_proj_t(edge_attr.T, r_weight).T
    return (edge_emb, rel_emb)
